# Initial kernel scaffold; baseline (speedup 1.0000x reference)
import jax
import jax.numpy as jnp
from jax.experimental import pallas as pl


def kernel(atom_feat, batch, W1, b1, W2, b2, W3, b3):
    raise NotImplementedError("write your pallas kernel here")


# speedup vs baseline: 4.9325x; 4.9325x over previous
import jax
import jax.numpy as jnp
from jax.experimental import pallas as pl
from jax.experimental.pallas import tpu as pltpu
from functools import partial

N = 100000
D = 128
G = 1024
H1 = 256
H2 = 128
CHUNK = 2000
NSTEPS = N // CHUNK


def _pool_mlp_kernel(x_ref, ids_ref, w1_ref, b1_ref, w2_ref, b2_ref,
                     w3_ref, b3_ref, out_ref, acc_ref):
    i = pl.program_id(0)

    @pl.when(i == 0)
    def _():
        acc_ref[...] = jnp.zeros_like(acc_ref)

    ids = ids_ref[0, 0, :]
    seg_iota = jax.lax.broadcasted_iota(jnp.int32, (G, CHUNK), 0)
    onehot_t = (seg_iota == ids[None, :]).astype(jnp.bfloat16)
    x = x_ref[...].astype(jnp.bfloat16)
    acc_ref[...] += jnp.dot(onehot_t, x, preferred_element_type=jnp.float32)

    @pl.when(i == NSTEPS - 1)
    def _():
        g = acc_ref[...]
        h = jnp.maximum(
            jnp.dot(g, w1_ref[...], preferred_element_type=jnp.float32)
            + b1_ref[...], 0.0)
        h = jnp.maximum(
            jnp.dot(h, w2_ref[...], preferred_element_type=jnp.float32)
            + b2_ref[...], 0.0)
        out_ref[...] = (
            jnp.dot(h, w3_ref[...], preferred_element_type=jnp.float32)
            + b3_ref[...])


@jax.jit
def kernel(atom_feat, batch, W1, b1, W2, b2, W3, b3):
    ids3 = batch.astype(jnp.int32).reshape(NSTEPS, 1, CHUNK)
    out = pl.pallas_call(
        _pool_mlp_kernel,
        grid=(NSTEPS,),
        in_specs=[
            pl.BlockSpec((CHUNK, D), lambda i: (i, 0)),
            pl.BlockSpec((1, 1, CHUNK), lambda i: (i, 0, 0)),
            pl.BlockSpec((D, H1), lambda i: (0, 0)),
            pl.BlockSpec((1, H1), lambda i: (0, 0)),
            pl.BlockSpec((H1, H2), lambda i: (0, 0)),
            pl.BlockSpec((1, H2), lambda i: (0, 0)),
            pl.BlockSpec((H2, 1), lambda i: (0, 0)),
            pl.BlockSpec((1, 1), lambda i: (0, 0)),
        ],
        out_specs=pl.BlockSpec((G, 1), lambda i: (0, 0)),
        out_shape=jax.ShapeDtypeStruct((G, 1), jnp.float32),
        scratch_shapes=[pltpu.VMEM((G, D), jnp.float32)],
        compiler_params=pltpu.CompilerParams(
            dimension_semantics=("arbitrary",)),
    )(atom_feat, ids3, W1, b1.reshape(1, H1), W2, b2.reshape(1, H2),
      W3, b3.reshape(1, 1))
    return out


# speedup vs baseline: 5.8514x; 1.1863x over previous
import jax
import jax.numpy as jnp
from jax import lax
from jax.experimental import pallas as pl
from jax.experimental.pallas import tpu as pltpu
from jax.experimental.pallas import tpu_sc as plsc
from functools import partial

N = 100000
D = 128
G = 1024
H1 = 256
H2 = 128

NC = 2
NS = 16
NW = NC * NS
BLK = 128
NBLK = 24
MAIN = NW * NBLK * BLK
TAILP = NW * BLK
GROWS = G // NS


def _sc_pool(x_hbm, ids2_hbm, tailx_hbm, tailids2_hbm, out_hbm,
             rows_v, idx_v, acc_sh, sem0, sem1):
    c = lax.axis_index("c")
    s = lax.axis_index("s")
    w = c * NS + s

    zz = jnp.zeros((16,), jnp.float32)

    @pl.loop(0, GROWS)
    def _(r):
        @pl.loop(0, D, step=16)
        def _(j):
            rows_v[0, r, pl.ds(j, 16)] = zz

    pltpu.sync_copy(rows_v.at[0, pl.ds(0, GROWS)],
                    acc_sh.at[pl.ds(s * GROWS, GROWS)])
    plsc.subcore_barrier()

    pltpu.sync_copy(ids2_hbm.at[pl.ds(w * NBLK, NBLK)],
                    idx_v.at[pl.ds(0, NBLK)])
    pltpu.sync_copy(tailids2_hbm.at[w], idx_v.at[NBLK])

    base = w * NBLK
    sems = (sem0, sem1)
    handles = [None, None]
    handles[0] = pltpu.async_copy(x_hbm.at[pl.ds(base * BLK, BLK)],
                                  rows_v.at[0], sem0)
    for i in range(NBLK + 1):
        buf = i % 2
        nxt = (i + 1) % 2
        if i + 1 < NBLK:
            handles[nxt] = pltpu.async_copy(
                x_hbm.at[pl.ds((base + i + 1) * BLK, BLK)],
                rows_v.at[nxt], sems[nxt])
        elif i + 1 == NBLK:
            handles[nxt] = pltpu.async_copy(
                tailx_hbm.at[pl.ds(w * BLK, BLK)], rows_v.at[nxt], sems[nxt])
        handles[buf].wait()
        pltpu.sync_copy(rows_v.at[buf], acc_sh.at[idx_v.at[i]], add=True)

    plsc.subcore_barrier()
    pltpu.sync_copy(acc_sh.at[pl.ds(s * GROWS, GROWS)],
                    out_hbm.at[c, pl.ds(s * GROWS, GROWS)])


def _mlp_kernel(p_ref, w1_ref, b1_ref, w2_ref, b2_ref, w3_ref, b3_ref,
                out_ref):
    g = p_ref[0] + p_ref[1]
    h = jnp.maximum(
        jnp.dot(g, w1_ref[...], preferred_element_type=jnp.float32)
        + b1_ref[...], 0.0)
    h = jnp.maximum(
        jnp.dot(h, w2_ref[...], preferred_element_type=jnp.float32)
        + b2_ref[...], 0.0)
    out_ref[...] = (
        jnp.dot(h, w3_ref[...], preferred_element_type=jnp.float32)
        + b3_ref[...])


@jax.jit
def kernel(atom_feat, batch, W1, b1, W2, b2, W3, b3):
    ids = batch.astype(jnp.int32)
    ids2 = ids[:MAIN].reshape(MAIN // BLK, BLK)
    ntail = N - MAIN
    tailx = jnp.zeros((TAILP, D), jnp.float32).at[:ntail].set(atom_feat[MAIN:])
    tailids = (jnp.zeros((TAILP,), jnp.int32).at[:ntail].set(ids[MAIN:])
               .reshape(NW, BLK))

    mesh = plsc.VectorSubcoreMesh(core_axis_name="c", subcore_axis_name="s")
    sc_pool = partial(
        pl.kernel,
        mesh=mesh,
        out_type=jax.ShapeDtypeStruct((NC, G, D), jnp.float32),
        scratch_types=[
            pltpu.VMEM((2, BLK, D), jnp.float32),
            pltpu.VMEM((NBLK + 1, BLK), jnp.int32),
            pltpu.VMEM_SHARED((G, D), jnp.float32),
            pltpu.SemaphoreType.DMA,
            pltpu.SemaphoreType.DMA,
        ],
    )(_sc_pool)
    partials = sc_pool(atom_feat, ids2, tailx, tailids)

    out = pl.pallas_call(
        _mlp_kernel,
        out_shape=jax.ShapeDtypeStruct((G, 1), jnp.float32),
    )(partials, W1, b1.reshape(1, H1), W2, b2.reshape(1, H2),
      W3, b3.reshape(1, 1))
    return out
